# baseline (speedup 1.0000x reference)
import jax
import jax.numpy as jnp
from jax import lax
from jax.experimental import pallas as pl
from jax.experimental.pallas import tpu as pltpu
from jax.experimental.pallas import tpu_sc as plsc

B = 16384
K = 32
NR = 2
W = 128
NC, NS, L = 2, 16, 16
NW = NC * NS
RPW = B // NW
CHK = 128

_LN2 = 0.6931471805599453
_SQRT2 = 1.4142135623730951


def _vlog(a):
    ab = lax.bitcast_convert_type(a, jnp.int32)
    e = lax.shift_right_logical(ab, 23) - 127
    m = lax.bitcast_convert_type(
        jnp.bitwise_or(jnp.bitwise_and(ab, 0x007FFFFF), 0x3F800000),
        jnp.float32)
    big = m > _SQRT2
    m = jnp.where(big, m * 0.5, m)
    ef = (e + jnp.where(big, 1, 0)).astype(jnp.float32)
    t = (m - 1.0) / (m + 1.0)
    t2 = t * t
    p = 2.0 + t2 * (2.0 / 3.0 + t2 * (2.0 / 5.0 + t2 * (2.0 / 7.0 + t2 * (2.0 / 9.0))))
    return ef * _LN2 + t * p


def _rule_neg_log(s):
    sig = 1.0 / (1.0 + jnp.exp(-s))
    return _vlog((1.0 - sig) + 1e-40)


def _body(users_r, items_r, gu_tab, gi_tab, xui_o, gu_o, gi_o,
          idx_u, idx_i, gu_v, gi_v, xui_v, sem_g, sem_o):
    wid = lax.axis_index("s") * NC + lax.axis_index("c")
    base = wid * RPW

    pltpu.sync_copy(users_r.at[pl.ds(base, RPW)], idx_u)
    pltpu.sync_copy(items_r.at[pl.ds(base, RPW)], idx_i)

    iota = lax.iota(jnp.int32, L)

    def chunk(c, carry0):
        cu = pltpu.async_copy(gu_tab.at[idx_u.at[pl.ds(c * CHK, CHK)]], gu_v, sem_g)
        ci = pltpu.async_copy(gi_tab.at[idx_i.at[pl.ds(c * CHK, CHK)]], gi_v, sem_g)
        cu.wait()
        ci.wait()

        def group(g, carry):
            def rowfn(r, accs):
                a0, a1 = accs
                b = g * L + r
                ia = gi_v[b, pl.ds(0, L)]
                ib = gi_v[b, pl.ds(L, L)]
                u0a = gu_v[b, pl.ds(0, L)]
                u0b = gu_v[b, pl.ds(L, L)]
                u1a = gu_v[b, pl.ds(2 * L, L)]
                u1b = gu_v[b, pl.ds(3 * L, L)]
                s0 = jnp.sum(u0a * ia + u0b * ib)
                s1 = jnp.sum(u1a * ia + u1b * ib)
                sel = iota == r
                return (jnp.where(sel, s0, a0), jnp.where(sel, s1, a1))

            z = jnp.zeros((L,), jnp.float32)
            a0, a1 = lax.fori_loop(0, L, rowfn, (z, z))
            log_sum = _rule_neg_log(a0) + _rule_neg_log(a1)
            xui_v[pl.ds(c * CHK + g * L, L)] = 1.0 - (-1.0 / (-1.0 + log_sum))
            return carry

        lax.fori_loop(0, CHK // L, group, 0)
        pltpu.sync_copy(gu_v, gu_o.at[pl.ds(base + c * CHK, CHK)])
        pltpu.sync_copy(gi_v, gi_o.at[pl.ds(base + c * CHK, CHK)])
        return carry0

    lax.fori_loop(0, RPW // CHK, chunk, 0)
    pltpu.sync_copy(xui_v, xui_o.at[pl.ds(base, RPW)])


def kernel(users, items, Gu, Gi):
    users = users.astype(jnp.int32)
    items = items.astype(jnp.int32)
    gu_p = jnp.pad(Gu, ((0, 0), (0, W - NR * K))) + 1e-45
    gi_p = jnp.pad(Gi, ((0, 0), (0, W - K))) + 1e-45
    run = pl.kernel(
        _body,
        out_type=(
            jax.ShapeDtypeStruct((B,), jnp.float32),
            jax.ShapeDtypeStruct((B, W), jnp.float32),
            jax.ShapeDtypeStruct((B, W), jnp.float32),
        ),
        mesh=plsc.VectorSubcoreMesh(core_axis_name="c", subcore_axis_name="s"),
        scratch_types=(
            pltpu.VMEM((RPW,), jnp.int32),
            pltpu.VMEM((RPW,), jnp.int32),
            pltpu.VMEM((CHK, W), jnp.float32),
            pltpu.VMEM((CHK, W), jnp.float32),
            pltpu.VMEM((RPW,), jnp.float32),
            pltpu.SemaphoreType.DMA,
            pltpu.SemaphoreType.DMA,
        ),
        compiler_params=pltpu.CompilerParams(
            needs_layout_passes=False, use_tc_tiling_on_sc=True),
    )
    xui, gu_pad, gi_pad = run(users, items, gu_p, gi_p)
    return xui, gu_pad[:, :NR * K].reshape(B, NR, K), gi_pad[:, :K]


# speedup vs baseline: 1.0412x; 1.0412x over previous
import jax
import jax.numpy as jnp
from jax import lax
from jax.experimental import pallas as pl
from jax.experimental.pallas import tpu as pltpu
from jax.experimental.pallas import tpu_sc as plsc

B = 16384
K = 32
NR = 2
W = 128
NC, NS, L = 2, 16, 16
NW = NC * NS
RPW = B // NW
CHK = 128

_LN2 = 0.6931471805599453
_SQRT2 = 1.4142135623730951


def _vlog(a):
    ab = lax.bitcast_convert_type(a, jnp.int32)
    e = lax.shift_right_logical(ab, 23) - 127
    m = lax.bitcast_convert_type(
        jnp.bitwise_or(jnp.bitwise_and(ab, 0x007FFFFF), 0x3F800000),
        jnp.float32)
    big = m > _SQRT2
    m = jnp.where(big, m * 0.5, m)
    ef = (e + jnp.where(big, 1, 0)).astype(jnp.float32)
    t = (m - 1.0) / (m + 1.0)
    t2 = t * t
    p = 2.0 + t2 * (2.0 / 3.0 + t2 * (2.0 / 5.0 + t2 * (2.0 / 7.0 + t2 * (2.0 / 9.0))))
    return ef * _LN2 + t * p


def _rule_neg_log(s):
    sig = 1.0 / (1.0 + jnp.exp(-s))
    return _vlog((1.0 - sig) + 1e-40)


def _body(users_r, items_r, gu_tab, gi_tab, xui_o, gu_o, gi_o,
          idx_u, idx_i, gu_v, gi_v, xui_v, sem_g, sem_o):
    wid = lax.axis_index("s") * NC + lax.axis_index("c")
    base = wid * RPW

    pltpu.sync_copy(users_r.at[pl.ds(base, RPW)], idx_u)
    pltpu.sync_copy(items_r.at[pl.ds(base, RPW)], idx_i)

    iota = lax.iota(jnp.int32, L)

    def chunk(c, carry0):
        cu = pltpu.async_copy(gu_tab.at[idx_u.at[pl.ds(c * CHK, CHK)]], gu_v, sem_g)
        ci = pltpu.async_copy(gi_tab.at[idx_i.at[pl.ds(c * CHK, CHK)]], gi_v, sem_g)
        cu.wait()
        ci.wait()

        def group(g, carry):
            def rowfn(r, accs):
                a0, a1 = accs
                b = g * L + r
                ia = gi_v[b, pl.ds(0, L)]
                ib = gi_v[b, pl.ds(L, L)]
                u0a = gu_v[b, pl.ds(0, L)]
                u0b = gu_v[b, pl.ds(L, L)]
                u1a = gu_v[b, pl.ds(2 * L, L)]
                u1b = gu_v[b, pl.ds(3 * L, L)]
                s0 = jnp.sum(u0a * ia + u0b * ib)
                s1 = jnp.sum(u1a * ia + u1b * ib)
                sel = iota == r
                return (jnp.where(sel, s0, a0), jnp.where(sel, s1, a1))

            z = jnp.zeros((L,), jnp.float32)
            a0, a1 = lax.fori_loop(0, L, rowfn, (z, z))
            log_sum = _rule_neg_log(a0) + _rule_neg_log(a1)
            xui_v[pl.ds(c * CHK + g * L, L)] = 1.0 - (-1.0 / (-1.0 + log_sum))
            return carry

        lax.fori_loop(0, CHK // L, group, 0)
        pltpu.sync_copy(gu_v, gu_o.at[pl.ds(base + c * CHK, CHK)])
        pltpu.sync_copy(gi_v, gi_o.at[pl.ds(base + c * CHK, CHK)])
        return carry0

    lax.fori_loop(0, RPW // CHK, chunk, 0)
    pltpu.sync_copy(xui_v, xui_o.at[pl.ds(base, RPW)])


def kernel(users, items, Gu, Gi):
    users = users.astype(jnp.int32)
    items = items.astype(jnp.int32)
    gu_p = jnp.pad(Gu, ((0, 0), (0, W - NR * K)))
    gi_p = jnp.pad(Gi, ((0, 0), (0, W - K)))
    run = pl.kernel(
        _body,
        out_type=(
            jax.ShapeDtypeStruct((B,), jnp.float32),
            jax.ShapeDtypeStruct((B, W), jnp.float32),
            jax.ShapeDtypeStruct((B, W), jnp.float32),
        ),
        mesh=plsc.VectorSubcoreMesh(core_axis_name="c", subcore_axis_name="s"),
        scratch_types=(
            pltpu.VMEM((RPW,), jnp.int32),
            pltpu.VMEM((RPW,), jnp.int32),
            pltpu.VMEM((CHK, W), jnp.float32),
            pltpu.VMEM((CHK, W), jnp.float32),
            pltpu.VMEM((RPW,), jnp.float32),
            pltpu.SemaphoreType.DMA,
            pltpu.SemaphoreType.DMA,
        ),
        compiler_params=pltpu.CompilerParams(
            needs_layout_passes=False, use_tc_tiling_on_sc=True,
            skip_device_barrier=True),
    )
    xui, gu_pad, gi_pad = run(users, items, gu_p, gi_p)
    return xui, gu_pad[:, :NR * K].reshape(B, NR, K), gi_pad[:, :K]
